# baseline (speedup 1.0000x reference)
import functools

import jax
import jax.numpy as jnp
from jax import lax
from jax.experimental import pallas as pl
from jax.experimental.pallas import tpu as pltpu
from jax.experimental.pallas import tpu_sc as plsc

_C = 19
_BH = 256
_NB = 32


def _stage1_body(x_ref, s1_ref, s2p_ref):
    x0 = x_ref[0, 0]
    m = x0
    s1 = x0
    s2 = x0 * x0
    for c in range(1, _C):
        xc = x_ref[0, c]
        m = jnp.maximum(m, xc)
        s1 = s1 + xc
        s2 = s2 + xc * xc
    pred = jnp.full(m.shape, _C - 1, jnp.int32)
    for c in range(_C - 2, -1, -1):
        pred = jnp.where(x_ref[0, c] == m, c, pred)
    lane = lax.broadcasted_iota(jnp.int32, m.shape, 1)
    idx = pred + (lane & 15) * _NB
    s2_bits = lax.bitcast_convert_type(s2, jnp.int32)
    s1_ref[...] = s1
    s2p_ref[...] = (s2_bits & ~511) | idx


def _stage2_body(s1_hbm, s2p_hbm, out_hbm, s1_v, s2p_v, b0, b1, b2):
    nc = 2
    wid = lax.axis_index("s") * nc + lax.axis_index("c")
    rows = s1_v.shape[0]
    base = wid * rows
    pltpu.sync_copy(s1_hbm.at[pl.ds(base, rows), :], s1_v)
    pltpu.sync_copy(s2p_hbm.at[pl.ds(base, rows), :], s2p_v)

    zero = jnp.zeros((16,), jnp.float32)
    for k in range(16 * _NB // 16):
        b0[pl.ds(k * 16, 16)] = zero
        b1[pl.ds(k * 16, 16)] = zero
        b2[pl.ds(k * 16, 16)] = zero

    ones = jnp.ones((16,), jnp.float32)
    mask_lo = jnp.full((16,), 511, jnp.int32)
    mask_hi = jnp.full((16,), ~511, jnp.int32)

    @plsc.parallel_loop(0, rows * 512, 16, unroll=8)
    def _(e):
        av = s1_v[e // 512, pl.ds(e % 512, 16)]
        pv = s2p_v[e // 512, pl.ds(e % 512, 16)]
        iv = pv & mask_lo
        bv = plsc.bitcast(pv & mask_hi, jnp.float32)
        plsc.addupdate_scatter(b0, [iv], ones)
        plsc.addupdate_scatter(b1, [iv], av)
        plsc.addupdate_scatter(b2, [iv], bv)

    pltpu.sync_copy(b0, out_hbm.at[wid, pl.ds(0, 512)])
    pltpu.sync_copy(b1, out_hbm.at[wid, pl.ds(512, 512)])
    pltpu.sync_copy(b2, out_hbm.at[wid, pl.ds(1024, 512)])


def _fold_lanes(x):
    acc = x[:, 0:_NB]
    for l in range(1, 16):
        acc = acc + x[:, l * _NB : (l + 1) * _NB]
    return acc


def _stage3_body(b_ref, out_ref, *, n):
    total = jnp.zeros((), jnp.float32)
    tiles_per_n = 32 // n
    for nn in range(n):
        s = jnp.sum(b_ref[nn * tiles_per_n : (nn + 1) * tiles_per_n, :], axis=0,
                    keepdims=True)
        cnt = _fold_lanes(s[:, 0:512]) * float(_C)
        s1 = _fold_lanes(s[:, 512:1024])
        s2 = _fold_lanes(s[:, 1024:1536])
        norms = jnp.sqrt(s2 - s1 * s1 / cnt)
        valid = lax.broadcasted_iota(jnp.int32, (1, _NB), 1) < _C
        total = total + jnp.sum(jnp.where(valid, norms, 0.0))
    out_ref[0, 0] = total / n


def kernel(logits, target):
    del target
    n, c, hh, w = logits.shape
    nh = hh // _BH
    s1, s2p = pl.pallas_call(
        _stage1_body,
        grid=(n, nh),
        in_specs=[pl.BlockSpec((1, c, _BH, w), lambda i, j: (i, 0, j, 0))],
        out_specs=[
            pl.BlockSpec((_BH, w), lambda i, j: (i * nh + j, 0)),
            pl.BlockSpec((_BH, w), lambda i, j: (i * nh + j, 0)),
        ],
        out_shape=[
            jax.ShapeDtypeStruct((n * hh, w), jnp.float32),
            jax.ShapeDtypeStruct((n * hh, w), jnp.int32),
        ],
    )(logits)

    rows = n * hh // 32
    stage2 = pl.kernel(
        _stage2_body,
        out_type=jax.ShapeDtypeStruct((32, 3 * 512), jnp.float32),
        mesh=plsc.VectorSubcoreMesh(core_axis_name="c", subcore_axis_name="s"),
        compiler_params=pltpu.CompilerParams(needs_layout_passes=False),
        scratch_types=[
            pltpu.VMEM((rows, w), jnp.float32),
            pltpu.VMEM((rows, w), jnp.int32),
            pltpu.VMEM((16 * _NB,), jnp.float32),
            pltpu.VMEM((16 * _NB,), jnp.float32),
            pltpu.VMEM((16 * _NB,), jnp.float32),
        ],
    )
    bins = stage2(s1, s2p)

    out = pl.pallas_call(
        functools.partial(_stage3_body, n=n),
        out_specs=pl.BlockSpec(memory_space=pltpu.SMEM),
        out_shape=jax.ShapeDtypeStruct((1, 1), jnp.float32),
    )(bins)
    return out[0, 0]


# speedup vs baseline: 1.0124x; 1.0124x over previous
import functools

import jax
import jax.numpy as jnp
from jax import lax
from jax.experimental import pallas as pl
from jax.experimental.pallas import tpu as pltpu
from jax.experimental.pallas import tpu_sc as plsc

_C = 19
_BH = 256
_NB = 32


def _stage1_body(x_ref, s1_ref, s2p_ref):
    x0 = x_ref[0, 0]
    m = x0
    s1 = x0
    s2 = x0 * x0
    for c in range(1, _C):
        xc = x_ref[0, c]
        m = jnp.maximum(m, xc)
        s1 = s1 + xc
        s2 = s2 + xc * xc
    pred = jnp.full(m.shape, _C - 1, jnp.int32)
    for c in range(_C - 2, -1, -1):
        pred = jnp.where(x_ref[0, c] == m, c, pred)
    lane = lax.broadcasted_iota(jnp.int32, m.shape, 1)
    idx = pred + (lane & 15) * _NB
    s2_bits = lax.bitcast_convert_type(s2, jnp.int32)
    s1_ref[...] = s1
    s2p_ref[...] = (s2_bits & ~511) | idx


def _stage2_body(s1_hbm, s2p_hbm, out_hbm, s1_v, s2p_v, b0, b1, b2,
                 sem_a, sem_b):
    nc = 2
    wid = lax.axis_index("s") * nc + lax.axis_index("c")
    rows = s1_v.shape[0]
    half = rows // 2
    base = wid * rows
    ca1 = pltpu.async_copy(s1_hbm.at[pl.ds(base, half), :],
                           s1_v.at[pl.ds(0, half), :], sem_a)
    ca2 = pltpu.async_copy(s2p_hbm.at[pl.ds(base, half), :],
                           s2p_v.at[pl.ds(0, half), :], sem_a)
    cb1 = pltpu.async_copy(s1_hbm.at[pl.ds(base + half, half), :],
                           s1_v.at[pl.ds(half, half), :], sem_b)
    cb2 = pltpu.async_copy(s2p_hbm.at[pl.ds(base + half, half), :],
                           s2p_v.at[pl.ds(half, half), :], sem_b)

    zero = jnp.zeros((16,), jnp.float32)
    for k in range(16 * _NB // 16):
        b0[pl.ds(k * 16, 16)] = zero
        b1[pl.ds(k * 16, 16)] = zero
        b2[pl.ds(k * 16, 16)] = zero

    ones = jnp.ones((16,), jnp.float32)
    mask_lo = jnp.full((16,), 511, jnp.int32)
    mask_hi = jnp.full((16,), ~511, jnp.int32)

    def scatter_span(lo, hi):
        @plsc.parallel_loop(lo, hi, 16, unroll=8)
        def _(e):
            av = s1_v[e // 512, pl.ds(e % 512, 16)]
            pv = s2p_v[e // 512, pl.ds(e % 512, 16)]
            iv = pv & mask_lo
            bv = plsc.bitcast(pv & mask_hi, jnp.float32)
            plsc.addupdate_scatter(b0, [iv], ones)
            plsc.addupdate_scatter(b1, [iv], av)
            plsc.addupdate_scatter(b2, [iv], bv)

    ca1.wait()
    ca2.wait()
    scatter_span(0, half * 512)
    cb1.wait()
    cb2.wait()
    scatter_span(half * 512, rows * 512)

    pltpu.sync_copy(b0, out_hbm.at[wid, pl.ds(0, 512)])
    pltpu.sync_copy(b1, out_hbm.at[wid, pl.ds(512, 512)])
    pltpu.sync_copy(b2, out_hbm.at[wid, pl.ds(1024, 512)])


def _fold_lanes(x):
    acc = x[:, 0:_NB]
    for l in range(1, 16):
        acc = acc + x[:, l * _NB : (l + 1) * _NB]
    return acc


def _stage3_body(b_ref, out_ref, *, n):
    total = jnp.zeros((), jnp.float32)
    tiles_per_n = 32 // n
    for nn in range(n):
        s = jnp.sum(b_ref[nn * tiles_per_n : (nn + 1) * tiles_per_n, :], axis=0,
                    keepdims=True)
        cnt = _fold_lanes(s[:, 0:512]) * float(_C)
        s1 = _fold_lanes(s[:, 512:1024])
        s2 = _fold_lanes(s[:, 1024:1536])
        norms = jnp.sqrt(s2 - s1 * s1 / cnt)
        valid = lax.broadcasted_iota(jnp.int32, (1, _NB), 1) < _C
        total = total + jnp.sum(jnp.where(valid, norms, 0.0))
    out_ref[0, 0] = total / n


def kernel(logits, target):
    del target
    n, c, hh, w = logits.shape
    nh = hh // _BH
    s1, s2p = pl.pallas_call(
        _stage1_body,
        grid=(n, nh),
        in_specs=[pl.BlockSpec((1, c, _BH, w), lambda i, j: (i, 0, j, 0))],
        out_specs=[
            pl.BlockSpec((_BH, w), lambda i, j: (i * nh + j, 0)),
            pl.BlockSpec((_BH, w), lambda i, j: (i * nh + j, 0)),
        ],
        out_shape=[
            jax.ShapeDtypeStruct((n * hh, w), jnp.float32),
            jax.ShapeDtypeStruct((n * hh, w), jnp.int32),
        ],
    )(logits)

    rows = n * hh // 32
    stage2 = pl.kernel(
        _stage2_body,
        out_type=jax.ShapeDtypeStruct((32, 3 * 512), jnp.float32),
        mesh=plsc.VectorSubcoreMesh(core_axis_name="c", subcore_axis_name="s"),
        compiler_params=pltpu.CompilerParams(needs_layout_passes=False),
        scratch_types=[
            pltpu.VMEM((rows, w), jnp.float32),
            pltpu.VMEM((rows, w), jnp.int32),
            pltpu.VMEM((16 * _NB,), jnp.float32),
            pltpu.VMEM((16 * _NB,), jnp.float32),
            pltpu.VMEM((16 * _NB,), jnp.float32),
            pltpu.SemaphoreType.DMA,
            pltpu.SemaphoreType.DMA,
        ],
    )
    bins = stage2(s1, s2p)

    out = pl.pallas_call(
        functools.partial(_stage3_body, n=n),
        out_specs=pl.BlockSpec(memory_space=pltpu.SMEM),
        out_shape=jax.ShapeDtypeStruct((1, 1), jnp.float32),
    )(bins)
    return out[0, 0]
